# Initial kernel scaffold; baseline (speedup 1.0000x reference)
import jax
import jax.numpy as jnp
from jax.experimental import pallas as pl


def kernel(center, context, negatives, in_embed, out_embed):
    raise NotImplementedError("write your pallas kernel here")


# speedup vs baseline: 4.7945x; 4.7945x over previous
import functools

import jax
import jax.numpy as jnp
from jax import lax
from jax.experimental import pallas as pl
from jax.experimental.pallas import tpu as pltpu
from jax.experimental.pallas import tpu_sc as plsc

_VOCAB = 1000000
_EMB = 64
_BATCH = 16384
_KNEG = 20

_NC = 2
_NS = 16
_NW = _NC * _NS
_EPW = _BATCH // _NW
_CB = 64
_NCHUNK = _EPW // _CB
_RPC = _CB * _KNEG
_IDX_DMA = 128
_NDMA = _RPC // _IDX_DMA


def _scores_call(center, context, negflat, in_embed, out_embed):
  mesh = plsc.VectorSubcoreMesh(core_axis_name="c", subcore_axis_name="s")

  @functools.partial(
      pl.kernel,
      out_type=[
          jax.ShapeDtypeStruct((_BATCH,), jnp.float32),
          jax.ShapeDtypeStruct((_BATCH * _KNEG,), jnp.float32),
      ],
      mesh=mesh,
      scratch_types=[
          pltpu.VMEM((_EPW,), jnp.int32),
          pltpu.VMEM((_EPW,), jnp.int32),
          pltpu.VMEM((_EPW * _KNEG,), jnp.int32),
          pltpu.VMEM((_CB, _EMB), jnp.float32),
          pltpu.VMEM((_CB, _EMB), jnp.float32),
          pltpu.VMEM((_RPC, _EMB), jnp.float32),
          pltpu.VMEM((_CB,), jnp.float32),
          pltpu.VMEM((_RPC,), jnp.float32),
          pltpu.SemaphoreType.DMA,
      ],
      compiler_params=pltpu.CompilerParams(
          needs_layout_passes=False, use_tc_tiling_on_sc=False),
  )
  def scores(cen_hbm, ctx_hbm, neg_hbm, in_hbm, out_hbm,
             pos_out, neg_out,
             idxc, idxo, idxn, crows, orows, nrows, psc, nsc, sem):
    wid = lax.axis_index("s") * _NC + lax.axis_index("c")
    base = wid * _EPW
    lanes = lax.iota(jnp.int32, 16)
    last_lane = lanes == 15
    pltpu.sync_copy(cen_hbm.at[pl.ds(base, _EPW)], idxc)
    pltpu.sync_copy(ctx_hbm.at[pl.ds(base, _EPW)], idxo)
    pltpu.sync_copy(neg_hbm.at[pl.ds(base * _KNEG, _EPW * _KNEG)], idxn)

    def chunk_body(c, carry):
      off = c * _CB
      h1 = pltpu.async_copy(in_hbm.at[idxc.at[pl.ds(off, _CB)]], crows, sem)
      h2 = pltpu.async_copy(out_hbm.at[idxo.at[pl.ds(off, _CB)]], orows, sem)
      hs = []
      for j in range(_NDMA):
        hs.append(pltpu.async_copy(
            out_hbm.at[idxn.at[pl.ds(c * _RPC + j * _IDX_DMA, _IDX_DMA)]],
            nrows.at[pl.ds(j * _IDX_DMA, _IDX_DMA)], sem))
      h1.wait()
      h2.wait()
      for h in hs:
        h.wait()

      def b_body(b, carry2):
        vc = [crows[b, pl.ds(j * 16, 16)] for j in range(4)]
        vo = [orows[b, pl.ds(j * 16, 16)] for j in range(4)]
        s = vc[0] * vo[0] + vc[1] * vo[1] + vc[2] * vo[2] + vc[3] * vo[3]
        plsc.store_scatter(psc, [jnp.full((16,), b, jnp.int32)],
                           jnp.full((16,), jnp.sum(s)), mask=last_lane)
        for kk in range(_KNEG):
          r = b * _KNEG + kk
          nv = [nrows[r, pl.ds(j * 16, 16)] for j in range(4)]
          t = vc[0] * nv[0] + vc[1] * nv[1] + vc[2] * nv[2] + vc[3] * nv[3]
          plsc.store_scatter(nsc, [jnp.full((16,), r, jnp.int32)],
                             jnp.full((16,), jnp.sum(t)), mask=last_lane)
        return carry2

      lax.fori_loop(0, _CB, b_body, 0)
      pltpu.sync_copy(psc, pos_out.at[pl.ds(base + off, _CB)])
      pltpu.sync_copy(nsc, neg_out.at[pl.ds((base + off) * _KNEG, _RPC)])
      return carry

    lax.fori_loop(0, _NCHUNK, chunk_body, 0)

  return scores(center, context, negflat, in_embed, out_embed)


def _loss_call(pos2d, neg2d):
  def body(pos_ref, neg_ref, out_ref):
    p = pos_ref[...]
    n = neg_ref[...]

    def logsig(x):
      return jnp.minimum(x, 0.0) - jnp.log1p(jnp.exp(-jnp.abs(x)))

    tot = jnp.sum(logsig(p)) + jnp.sum(logsig(-n))
    out_ref[...] = jnp.full((1, 1), -tot / _BATCH, jnp.float32)

  return pl.pallas_call(
      body,
      out_shape=jax.ShapeDtypeStruct((1, 1), jnp.float32),
  )(pos2d, neg2d)


def kernel(center, context, negatives, in_embed, out_embed):
  c = center.astype(jnp.int32)
  o = context.astype(jnp.int32)
  n = negatives.astype(jnp.int32).reshape(-1)
  pos, neg = _scores_call(c, o, n, in_embed, out_embed)
  loss = _loss_call(pos.reshape(128, 128), neg.reshape(2560, 128))
  return loss[0, 0]


# speedup vs baseline: 7.7968x; 1.6262x over previous
import functools

import jax
import jax.numpy as jnp
from jax import lax
from jax.experimental import pallas as pl
from jax.experimental.pallas import tpu as pltpu
from jax.experimental.pallas import tpu_sc as plsc

_VOCAB = 1000000
_EMB = 64
_BATCH = 16384
_KNEG = 20

_NC = 2
_NS = 16
_NW = _NC * _NS
_EPW = _BATCH // _NW
_CB = 32
_NCHUNK = _EPW // _CB
_RPC = _CB * _KNEG
_IDX_DMA = 128
_NDMA = _RPC // _IDX_DMA

_BT = 4096
_NBT = -(-_VOCAB // _BT)


def _transpose_call(inT, outT):
  def body(x_ref, y_ref, ox_ref, oy_ref):
    ox_ref[...] = jnp.pad(jnp.transpose(x_ref[...]), ((0, 0), (0, 64)))
    oy_ref[...] = jnp.pad(jnp.transpose(y_ref[...]), ((0, 0), (0, 64)))

  return pl.pallas_call(
      body,
      grid=(_NBT,),
      in_specs=[
          pl.BlockSpec((_EMB, _BT), lambda j: (0, j)),
          pl.BlockSpec((_EMB, _BT), lambda j: (0, j)),
      ],
      out_specs=[
          pl.BlockSpec((_BT, 128), lambda j: (j, 0)),
          pl.BlockSpec((_BT, 128), lambda j: (j, 0)),
      ],
      out_shape=[
          jax.ShapeDtypeStruct((_VOCAB, 128), jnp.float32),
          jax.ShapeDtypeStruct((_VOCAB, 128), jnp.float32),
      ],
  )(inT, outT)


def _scores_call(center, context, negflat, in128, out128):
  mesh = plsc.VectorSubcoreMesh(core_axis_name="c", subcore_axis_name="s")

  @functools.partial(
      pl.kernel,
      out_type=[
          jax.ShapeDtypeStruct((_BATCH,), jnp.float32),
          jax.ShapeDtypeStruct((_BATCH * _KNEG,), jnp.float32),
      ],
      mesh=mesh,
      scratch_types=[
          pltpu.VMEM((_EPW,), jnp.int32),
          pltpu.VMEM((_EPW,), jnp.int32),
          pltpu.VMEM((_EPW * _KNEG,), jnp.int32),
          pltpu.VMEM((_CB, 128), jnp.float32),
          pltpu.VMEM((_CB, 128), jnp.float32),
          pltpu.VMEM((_RPC, 128), jnp.float32),
          pltpu.VMEM((_CB,), jnp.float32),
          pltpu.VMEM((_RPC,), jnp.float32),
          pltpu.SemaphoreType.DMA,
      ],
      compiler_params=pltpu.CompilerParams(
          needs_layout_passes=False, use_tc_tiling_on_sc=True),
  )
  def scores(cen_hbm, ctx_hbm, neg_hbm, in_hbm, out_hbm,
             pos_out, neg_out,
             idxc, idxo, idxn, crows, orows, nrows, psc, nsc, sem):
    wid = lax.axis_index("s") * _NC + lax.axis_index("c")
    base = wid * _EPW
    lanes = lax.iota(jnp.int32, 16)
    last_lane = lanes == 15
    pltpu.sync_copy(cen_hbm.at[pl.ds(base, _EPW)], idxc)
    pltpu.sync_copy(ctx_hbm.at[pl.ds(base, _EPW)], idxo)
    pltpu.sync_copy(neg_hbm.at[pl.ds(base * _KNEG, _EPW * _KNEG)], idxn)

    def chunk_body(c, carry):
      off = c * _CB
      h1 = pltpu.async_copy(in_hbm.at[idxc.at[pl.ds(off, _CB)]], crows, sem)
      h2 = pltpu.async_copy(out_hbm.at[idxo.at[pl.ds(off, _CB)]], orows, sem)
      hs = []
      for j in range(_NDMA):
        hs.append(pltpu.async_copy(
            out_hbm.at[idxn.at[pl.ds(c * _RPC + j * _IDX_DMA, _IDX_DMA)]],
            nrows.at[pl.ds(j * _IDX_DMA, _IDX_DMA)], sem))
      h1.wait()
      h2.wait()
      for h in hs:
        h.wait()

      def b_body(b, carry2):
        vc = [crows[b, pl.ds(j * 16, 16)] for j in range(4)]
        vo = [orows[b, pl.ds(j * 16, 16)] for j in range(4)]
        s = vc[0] * vo[0] + vc[1] * vo[1] + vc[2] * vo[2] + vc[3] * vo[3]
        plsc.store_scatter(psc, [jnp.full((16,), b, jnp.int32)],
                           jnp.full((16,), jnp.sum(s)), mask=last_lane)
        for kk in range(_KNEG):
          r = b * _KNEG + kk
          nv = [nrows[r, pl.ds(j * 16, 16)] for j in range(4)]
          t = vc[0] * nv[0] + vc[1] * nv[1] + vc[2] * nv[2] + vc[3] * nv[3]
          plsc.store_scatter(nsc, [jnp.full((16,), r, jnp.int32)],
                             jnp.full((16,), jnp.sum(t)), mask=last_lane)
        return carry2

      lax.fori_loop(0, _CB, b_body, 0)
      pltpu.sync_copy(psc, pos_out.at[pl.ds(base + off, _CB)])
      pltpu.sync_copy(nsc, neg_out.at[pl.ds((base + off) * _KNEG, _RPC)])
      return carry

    lax.fori_loop(0, _NCHUNK, chunk_body, 0)

  return scores(center, context, negflat, in128, out128)


def _loss_call(pos2d, neg2d):
  def body(pos_ref, neg_ref, out_ref):
    p = pos_ref[...]
    n = neg_ref[...]

    def logsig(x):
      return jnp.minimum(x, 0.0) - jnp.log1p(jnp.exp(-jnp.abs(x)))

    tot = jnp.sum(logsig(p)) + jnp.sum(logsig(-n))
    out_ref[...] = jnp.full((1, 1), -tot / _BATCH, jnp.float32)

  return pl.pallas_call(
      body,
      out_shape=jax.ShapeDtypeStruct((1, 1), jnp.float32),
  )(pos2d, neg2d)


def kernel(center, context, negatives, in_embed, out_embed):
  c = center.astype(jnp.int32)
  o = context.astype(jnp.int32)
  n = negatives.astype(jnp.int32).reshape(-1)
  in128, out128 = _transpose_call(in_embed.T, out_embed.T)
  pos, neg = _scores_call(c, o, n, in128, out128)
  loss = _loss_call(pos.reshape(128, 128), neg.reshape(2560, 128))
  return loss[0, 0]
